# baseline (speedup 1.0000x reference)
import functools

import jax
import jax.numpy as jnp
from jax import lax
from jax.experimental import pallas as pl
from jax.experimental.pallas import tpu as pltpu
from jax.experimental.pallas import tpu_sc as plsc

_N = 16384
_K = 64
_NC = 2
_NS = 16
_L = 16
_NW = _NC * _NS
_ROWS_W = _N // _NW
_CHUNK = _ROWS_W // 2
_GROUPS_C = _CHUNK // _L


def _lane_sum(v):
    idx = lax.iota(jnp.int32, _L)
    dnums = lax.GatherDimensionNumbers(
        offset_dims=(), collapsed_slice_dims=(0,), start_index_map=(0,))
    for s in (1, 2, 4, 8):
        perm = lax.gather(
            v, (idx ^ s)[:, None], dimension_numbers=dnums, slice_sizes=(1,),
            mode=lax.GatherScatterMode.PROMISE_IN_BOUNDS)
        v = v + perm
    return v


def _build():
    mesh = plsc.VectorSubcoreMesh(core_axis_name="c", subcore_axis_name="s")

    @functools.partial(
        pl.kernel,
        mesh=mesh,
        out_type=jax.ShapeDtypeStruct((_N,), jnp.float32),
        scratch_types=[
            pltpu.VMEM((_CHUNK, _K), jnp.float32),
            pltpu.VMEM((_CHUNK, _K), jnp.float32),
            pltpu.VMEM((_ROWS_W,), jnp.float32),
        ],
    )
    def sc_rowdot(gu_hbm, gi_hbm, out_hbm, gu_v, gi_v, out_v):
        wid = lax.axis_index("s") * _NC + lax.axis_index("c")
        base = wid * _ROWS_W
        lanes = lax.iota(jnp.int32, _L)

        def chunk_body(chunk, carry):
            cbase = base + chunk * _CHUNK
            pltpu.sync_copy(gu_hbm.at[pl.ds(cbase, _CHUNK), :], gu_v)
            pltpu.sync_copy(gi_hbm.at[pl.ds(cbase, _CHUNK), :], gi_v)

            for g in range(_GROUPS_C):
                acc_out = jnp.zeros((_L,), jnp.float32)
                for r in range(_L):
                    row = g * _L + r
                    acc = gu_v[row, pl.ds(0, _L)] * gi_v[row, pl.ds(0, _L)]
                    for c in range(1, _K // _L):
                        acc = acc + (gu_v[row, pl.ds(c * _L, _L)] *
                                     gi_v[row, pl.ds(c * _L, _L)])
                    s = _lane_sum(acc)
                    acc_out = jnp.where(lanes == r, s, acc_out)
                out_v[pl.ds(chunk * _CHUNK + g * _L, _L)] = acc_out
            return carry

        lax.fori_loop(0, _ROWS_W // _CHUNK, chunk_body, 0)

        pltpu.sync_copy(out_v, out_hbm.at[pl.ds(base, _ROWS_W)])

    return sc_rowdot


def kernel(gu, gi):
    return _build()(gu, gi)


# speedup vs baseline: 1.1207x; 1.1207x over previous
import functools

import jax
import jax.numpy as jnp
from jax import lax
from jax.experimental import pallas as pl
from jax.experimental.pallas import tpu as pltpu
from jax.experimental.pallas import tpu_sc as plsc

_N = 16384
_K = 64
_NC = 2
_NS = 16
_L = 16
_NW = _NC * _NS
_ROWS_W = _N // _NW
_CHUNK = _ROWS_W // 4
_NCHUNK = _ROWS_W // _CHUNK
_GROUPS_C = _CHUNK // _L


def _lane_sum(v):
    idx = lax.iota(jnp.int32, _L)
    dnums = lax.GatherDimensionNumbers(
        offset_dims=(), collapsed_slice_dims=(0,), start_index_map=(0,))
    for s in (1, 2, 4, 8):
        perm = lax.gather(
            v, (idx ^ s)[:, None], dimension_numbers=dnums, slice_sizes=(1,),
            mode=lax.GatherScatterMode.PROMISE_IN_BOUNDS)
        v = v + perm
    return v


def _build():
    mesh = plsc.VectorSubcoreMesh(core_axis_name="c", subcore_axis_name="s")

    @functools.partial(
        pl.kernel,
        mesh=mesh,
        out_type=jax.ShapeDtypeStruct((_N,), jnp.float32),
        scratch_types=[
            pltpu.VMEM((2, _CHUNK, _K), jnp.float32),
            pltpu.VMEM((2, _CHUNK, _K), jnp.float32),
            pltpu.VMEM((_ROWS_W,), jnp.float32),
            pltpu.SemaphoreType.DMA,
            pltpu.SemaphoreType.DMA,
        ],
    )
    def sc_rowdot(gu_hbm, gi_hbm, out_hbm, gu_v, gi_v, out_v, sem0, sem1):
        wid = lax.axis_index("s") * _NC + lax.axis_index("c")
        base = wid * _ROWS_W
        lanes = lax.iota(jnp.int32, _L)
        sems = (sem0, sem1)

        def copies(chunk, slot):
            cbase = base + chunk * _CHUNK
            return (
                pltpu.make_async_copy(
                    gu_hbm.at[pl.ds(cbase, _CHUNK), :], gu_v.at[slot],
                    sems[slot]),
                pltpu.make_async_copy(
                    gi_hbm.at[pl.ds(cbase, _CHUNK), :], gi_v.at[slot],
                    sems[slot]),
            )

        for c_ in copies(0, 0):
            c_.start()

        for chunk in range(_NCHUNK):
            slot = chunk % 2
            if chunk + 1 < _NCHUNK:
                for c_ in copies(chunk + 1, (chunk + 1) % 2):
                    c_.start()
            for c_ in copies(chunk, slot):
                c_.wait()

            def group(g, carry, chunk=chunk, slot=slot):
                acc_out = jnp.zeros((_L,), jnp.float32)
                for r in range(_L):
                    row = g * _L + r
                    acc = (gu_v[slot, row, pl.ds(0, _L)] *
                           gi_v[slot, row, pl.ds(0, _L)])
                    for c in range(1, _K // _L):
                        acc = acc + (gu_v[slot, row, pl.ds(c * _L, _L)] *
                                     gi_v[slot, row, pl.ds(c * _L, _L)])
                    s = _lane_sum(acc)
                    acc_out = jnp.where(lanes == r, s, acc_out)
                out_v[pl.ds(chunk * _CHUNK + g * _L, _L)] = acc_out
                return carry

            lax.fori_loop(0, _GROUPS_C, group, 0, unroll=4)

        pltpu.sync_copy(out_v, out_hbm.at[pl.ds(base, _ROWS_W)])

    return sc_rowdot


def kernel(gu, gi):
    return _build()(gu, gi)
